# baseline (speedup 1.0000x reference)
import functools
import math

import jax
import jax.numpy as jnp
from jax.experimental import pallas as pl
from jax.experimental.pallas import tpu as pltpu

D_MODEL_C = 768
N_FEAT_C = 24576
K_C = 64
N_TOK_C = 2048

TBLK = 256
FBLK = 1024


def _enc_body(x_ref, w_ref, b_ref, acts_ref):
    acts = jnp.dot(x_ref[...], w_ref[...], preferred_element_type=jnp.float32)
    acts = acts + b_ref[...]
    acts_ref[...] = jnp.maximum(acts, 0.0)


def _encoder_acts(x_n, W_enc, b_mid):
    n_tok, d = x_n.shape
    n_feat = W_enc.shape[1]
    grid = (n_tok // TBLK, n_feat // FBLK)
    return pl.pallas_call(
        _enc_body,
        grid=grid,
        in_specs=[
            pl.BlockSpec((TBLK, d), lambda i, j: (i, 0)),
            pl.BlockSpec((d, FBLK), lambda i, j: (0, j)),
            pl.BlockSpec((1, FBLK), lambda i, j: (0, j)),
        ],
        out_specs=pl.BlockSpec((TBLK, FBLK), lambda i, j: (i, j)),
        out_shape=jax.ShapeDtypeStruct((n_tok, n_feat), jnp.float32),
    )(x_n, W_enc, b_mid.reshape(1, -1))


def kernel(x, W_enc, b_mid, W_dec, b_pre, avg_norm):
    tgt_norm = math.sqrt(x.shape[1])
    x_n = x / avg_norm * tgt_norm - b_pre[None, :]
    acts = _encoder_acts(x_n, W_enc, b_mid)
    k_weights, k_indices = jax.lax.top_k(acts, K_C)
    y_n = jnp.einsum('bk,bkd->bd', k_weights, jnp.take(W_dec, k_indices, axis=0))
    y = (y_n + b_pre[None, :]) / tgt_norm * avg_norm
    return y, k_weights, k_indices


# speedup vs baseline: 1.1341x; 1.1341x over previous
import functools
import math

import jax
import jax.numpy as jnp
from jax import lax
from jax.experimental import pallas as pl
from jax.experimental.pallas import tpu as pltpu
from jax.experimental.pallas import tpu_sc as plsc

D_MODEL_C = 768
N_FEAT_C = 24576
K_C = 64
N_TOK_C = 2048

TBLK = 256
FBLK = 1024


def _enc_body(x_ref, w_ref, b_ref, acts_ref):
    acts = jnp.dot(x_ref[...], w_ref[...], preferred_element_type=jnp.float32)
    acts = acts + b_ref[...]
    acts_ref[...] = jnp.maximum(acts, 0.0)


def _encoder_acts(x_n, W_enc, b_mid):
    n_tok, d = x_n.shape
    n_feat = W_enc.shape[1]
    grid = (n_tok // TBLK, n_feat // FBLK)
    return pl.pallas_call(
        _enc_body,
        grid=grid,
        in_specs=[
            pl.BlockSpec((TBLK, d), lambda i, j: (i, 0)),
            pl.BlockSpec((d, FBLK), lambda i, j: (0, j)),
            pl.BlockSpec((1, FBLK), lambda i, j: (0, j)),
        ],
        out_specs=pl.BlockSpec((TBLK, FBLK), lambda i, j: (i, j)),
        out_shape=jax.ShapeDtypeStruct((n_tok, n_feat), jnp.float32),
    )(x_n, W_enc, b_mid.reshape(1, -1))


NW = 32
TPW = N_TOK_C // NW
NL = 16
DCH = D_MODEL_C // NL


def _decode_body(idx_hbm, w_hbm, wdec_hbm, out_hbm, idx_v, w_v, rows_v, y_v, sem):
    wid = lax.axis_index("s") * 2 + lax.axis_index("c")
    base = wid * TPW
    pltpu.sync_copy(idx_hbm.at[pl.ds(base, TPW)], idx_v)
    pltpu.sync_copy(w_hbm.at[pl.ds(base * K_C, TPW * K_C)], w_v)

    def tok_body(t, _):
        pltpu.async_copy(wdec_hbm.at[idx_v.at[t]], rows_v, sem).wait()

        def grp_body(g, acc):
            wvec = w_v[pl.ds(t * K_C + g * NL, NL)]
            for j0 in range(NL):
                w = wvec[j0]
                j = g * NL + j0
                acc = tuple(acc[c] + w * rows_v[j, pl.ds(c * NL, NL)]
                            for c in range(DCH))
            return acc

        zero = jnp.zeros((NL,), jnp.float32)
        acc = lax.fori_loop(0, K_C // NL, grp_body, (zero,) * DCH)
        for c in range(DCH):
            y_v[t, pl.ds(c * NL, NL)] = acc[c]
        return 0

    lax.fori_loop(0, TPW, tok_body, 0)
    pltpu.sync_copy(y_v, out_hbm.at[pl.ds(base, TPW)])


@functools.partial(jax.jit, static_argnames=())
def _sc_decode(k_indices, k_weights, W_dec):
    mesh = plsc.VectorSubcoreMesh(core_axis_name="c", subcore_axis_name="s")
    f = pl.kernel(
        _decode_body,
        out_type=jax.ShapeDtypeStruct((N_TOK_C, D_MODEL_C), jnp.float32),
        mesh=mesh,
        scratch_types=[
            pltpu.VMEM((TPW, K_C), jnp.int32),
            pltpu.VMEM((TPW * K_C,), jnp.float32),
            pltpu.VMEM((K_C, D_MODEL_C), jnp.float32),
            pltpu.VMEM((TPW, D_MODEL_C), jnp.float32),
            pltpu.SemaphoreType.DMA,
        ],
    )
    return f(k_indices, k_weights.reshape(-1), W_dec)


def kernel(x, W_enc, b_mid, W_dec, b_pre, avg_norm):
    tgt_norm = math.sqrt(x.shape[1])
    x_n = x / avg_norm * tgt_norm - b_pre[None, :]
    acts = _encoder_acts(x_n, W_enc, b_mid)
    k_weights, k_indices = jax.lax.top_k(acts, K_C)
    y_n = _sc_decode(k_indices, k_weights, W_dec)
    y = (y_n + b_pre[None, :]) / tgt_norm * avg_norm
    return y, k_weights, k_indices


# speedup vs baseline: 7.9100x; 6.9745x over previous
import functools
import math

import jax
import jax.numpy as jnp
from jax import lax
from jax.experimental import pallas as pl
from jax.experimental.pallas import tpu as pltpu
from jax.experimental.pallas import tpu_sc as plsc

D_MODEL_C = 768
N_FEAT_C = 24576
K_C = 64
N_TOK_C = 2048

TBLK = 256
FBLK = 1024
BPB = FBLK // 128
N_BLK = N_FEAT_C // 128
NBC = N_BLK // 16

NW = 32
TPW = N_TOK_C // NW
NL = 16
DCH = D_MODEL_C // NL

NEG = -1.0


def _enc_body(x_ref, w_ref, b_ref, acts_ref, bmax_ref):
    acts = jnp.dot(x_ref[...], w_ref[...], preferred_element_type=jnp.float32)
    acts = jnp.maximum(acts + b_ref[...], 0.0)
    acts_ref[...] = acts
    bm = jnp.concatenate(
        [jnp.max(acts[:, k * 128:(k + 1) * 128], axis=1, keepdims=True)
         for k in range(BPB)], axis=1)
    bmax_ref[...] = bm[None]


def _encoder_acts(x_n, W_enc, b_mid):
    grid = (N_TOK_C // TBLK, N_FEAT_C // FBLK)
    return pl.pallas_call(
        _enc_body,
        grid=grid,
        in_specs=[
            pl.BlockSpec((TBLK, D_MODEL_C), lambda i, j: (i, 0)),
            pl.BlockSpec((D_MODEL_C, FBLK), lambda i, j: (0, j)),
            pl.BlockSpec((1, FBLK), lambda i, j: (0, j)),
        ],
        out_specs=[
            pl.BlockSpec((TBLK, FBLK), lambda i, j: (i, j)),
            pl.BlockSpec((1, TBLK, BPB), lambda i, j: (j, i, 0)),
        ],
        out_shape=[
            jax.ShapeDtypeStruct((N_TOK_C, N_FEAT_C), jnp.float32),
            jax.ShapeDtypeStruct((N_FEAT_C // FBLK, N_TOK_C, BPB),
                                 jnp.float32),
        ],
    )(x_n, W_enc, b_mid.reshape(1, -1))


_GDN = lax.GatherDimensionNumbers(
    offset_dims=(), collapsed_slice_dims=(0,), start_index_map=(0,))


def _shuf(v, perm):
    return lax.gather(v, perm[:, None], _GDN, (1,),
                      mode=lax.GatherScatterMode.PROMISE_IN_BOUNDS)


def _perms():
    iota = lax.iota(jnp.int32, NL)
    return [iota ^ d for d in (1, 2, 4, 8)]


def _bmaxv(v, perms):
    for p in perms:
        v = jnp.maximum(v, _shuf(v, p))
    return v


def _bminv(v, perms):
    for p in perms:
        v = jnp.minimum(v, _shuf(v, p))
    return v


def _topk_body(bm_hbm, acts_hbm, kw_hbm, ki_hbm,
               bm_v, row_v, ow_v, oi_v, sem, semb):
    wid = lax.axis_index("s") * 2 + lax.axis_index("c")
    base = wid * TPW
    iota = lax.iota(jnp.int32, NL)
    perms = _perms()
    big = jnp.int32(1 << 24)

    def dma(t, buf):
        return pltpu.make_async_copy(
            acts_hbm.at[pl.ds(base + t, 1)], row_v.at[pl.ds(buf, 1)], sem)

    def dmab(t, buf):
        return pltpu.make_async_copy(
            bm_hbm.at[pl.ds(base + t, 1)], bm_v.at[pl.ds(buf, 1)], semb)

    dma(0, 0).start()
    dmab(0, 0).start()

    def tok_body(t, _):
        buf = lax.rem(t, 2)
        dma(t, buf).wait()
        dmab(t, buf).wait()

        @pl.when(t + 1 < TPW)
        def _():
            dma(t + 1, 1 - buf).start()
            dmab(t + 1, 1 - buf).start()

        def sel_body(k, _):
            m = bm_v[buf, pl.ds(0, NL)]
            for i in range(1, NBC):
                m = jnp.maximum(m, bm_v[buf, pl.ds(i * NL, NL)])
            mxv = _bmaxv(m, perms)
            mx = mxv[0]
            pos = jnp.zeros((NL,), jnp.int32) + big
            for i in range(NBC):
                c = bm_v[buf, pl.ds(i * NL, NL)]
                pos = jnp.minimum(
                    pos, jnp.where(c == mx, iota + i * NL, big))
            b = _bminv(pos, perms)[0]
            fbase = b * 128
            p2 = jnp.zeros((NL,), jnp.int32) + big
            for r in range(8):
                c = row_v[buf, pl.ds(fbase + r * NL, NL)]
                p2 = jnp.minimum(
                    p2, jnp.where(c == mx, iota + r * NL, big))
            off = _bminv(p2, perms)[0]
            feat = fbase + off
            rwin = off // NL
            lwin = off - rwin * NL
            ch = row_v[buf, pl.ds(fbase + rwin * NL, NL)]
            row_v[buf, pl.ds(fbase + rwin * NL, NL)] = jnp.where(
                iota == lwin, jnp.float32(NEG), ch)
            nm = jnp.zeros((NL,), jnp.float32) + NEG
            for r in range(8):
                nm = jnp.maximum(nm, row_v[buf, pl.ds(fbase + r * NL, NL)])
            nbm = _bmaxv(nm, perms)[0]
            bc = b // NL
            bl = b - bc * NL
            bch = bm_v[buf, pl.ds(bc * NL, NL)]
            bm_v[buf, pl.ds(bc * NL, NL)] = jnp.where(iota == bl, nbm, bch)
            og = (k // NL) * NL
            l2 = k - og
            wch = ow_v[pl.ds(og, NL)]
            ow_v[pl.ds(og, NL)] = jnp.where(iota == l2, mx, wch)
            ich = oi_v[pl.ds(og, NL)]
            oi_v[pl.ds(og, NL)] = jnp.where(iota == l2, feat, ich)
            return 0

        lax.fori_loop(0, K_C, sel_body, 0)
        pltpu.sync_copy(ow_v, kw_hbm.at[pl.ds((base + t) * K_C, K_C)])
        pltpu.sync_copy(oi_v, ki_hbm.at[pl.ds((base + t) * K_C, K_C)])
        return 0

    lax.fori_loop(0, TPW, tok_body, 0)


def _sc_topk(bmax, acts):
    mesh = plsc.VectorSubcoreMesh(core_axis_name="c", subcore_axis_name="s")
    f = pl.kernel(
        _topk_body,
        out_type=(jax.ShapeDtypeStruct((N_TOK_C * K_C,), jnp.float32),
                  jax.ShapeDtypeStruct((N_TOK_C * K_C,), jnp.int32)),
        mesh=mesh,
        scratch_types=[
            pltpu.VMEM((2, N_BLK), jnp.float32),
            pltpu.VMEM((2, N_FEAT_C), jnp.float32),
            pltpu.VMEM((K_C,), jnp.float32),
            pltpu.VMEM((K_C,), jnp.int32),
            pltpu.SemaphoreType.DMA,
            pltpu.SemaphoreType.DMA,
        ],
    )
    kw, ki = f(bmax, acts)
    return kw.reshape(N_TOK_C, K_C), ki.reshape(N_TOK_C, K_C)


def _decode_body(idx_hbm, w_hbm, wdec_hbm, out_hbm, idx_v, w_v, rows_v, y_v,
                 sem):
    wid = lax.axis_index("s") * 2 + lax.axis_index("c")
    base = wid * TPW
    pltpu.sync_copy(idx_hbm.at[pl.ds(base, TPW)], idx_v)
    pltpu.sync_copy(w_hbm.at[pl.ds(base * K_C, TPW * K_C)], w_v)

    def dma(t, buf):
        return pltpu.make_async_copy(
            wdec_hbm.at[idx_v.at[t]], rows_v.at[buf], sem)

    pltpu.async_copy(wdec_hbm.at[idx_v.at[0]], rows_v.at[0], sem).start()

    def tok_body(t, _):
        buf = lax.rem(t, 2)
        dma(t, buf).wait()

        @pl.when(t + 1 < TPW)
        def _():
            dma(t + 1, 1 - buf).start()

        def grp_body(g, acc):
            wvec = w_v[pl.ds(t * K_C + g * NL, NL)]
            for j0 in range(NL):
                w = wvec[j0]
                j = g * NL + j0
                acc = tuple(acc[c] + w * rows_v[buf, j, pl.ds(c * NL, NL)]
                            for c in range(DCH))
            return acc

        zero = jnp.zeros((NL,), jnp.float32)
        acc = lax.fori_loop(0, K_C // NL, grp_body, (zero,) * DCH)
        for c in range(DCH):
            y_v[pl.ds(c * NL, NL)] = acc[c]
        pltpu.sync_copy(
            y_v, out_hbm.at[pl.ds((base + t) * D_MODEL_C, D_MODEL_C)])
        return 0

    lax.fori_loop(0, TPW, tok_body, 0)


def _sc_decode(k_indices, k_weights, W_dec):
    mesh = plsc.VectorSubcoreMesh(core_axis_name="c", subcore_axis_name="s")
    f = pl.kernel(
        _decode_body,
        out_type=jax.ShapeDtypeStruct((N_TOK_C * D_MODEL_C,), jnp.float32),
        mesh=mesh,
        scratch_types=[
            pltpu.VMEM((TPW, K_C), jnp.int32),
            pltpu.VMEM((TPW * K_C,), jnp.float32),
            pltpu.VMEM((2, K_C, D_MODEL_C), jnp.float32),
            pltpu.VMEM((D_MODEL_C,), jnp.float32),
            pltpu.SemaphoreType.DMA,
        ],
    )
    y = f(k_indices, k_weights.reshape(-1), W_dec)
    return y.reshape(N_TOK_C, D_MODEL_C)


def kernel(x, W_enc, b_mid, W_dec, b_pre, avg_norm):
    tgt_norm = math.sqrt(x.shape[1])
    x_n = x / avg_norm * tgt_norm - b_pre[None, :]
    acts, bmax3 = _encoder_acts(x_n, W_enc, b_mid)
    bmax = bmax3.transpose(1, 0, 2).reshape(N_TOK_C, N_BLK)
    k_weights, k_indices = _sc_topk(bmax, acts)
    y_n = _sc_decode(k_indices, k_weights, W_dec)
    y = (y_n + b_pre[None, :]) / tgt_norm * avg_norm
    return y, k_weights, k_indices
